# baseline (speedup 1.0000x reference)
import functools

import jax
import jax.numpy as jnp
from jax import lax
from jax.experimental import pallas as pl
from jax.experimental.pallas import tpu as pltpu
from jax.experimental.pallas import tpu_sc as plsc

N = 10000
E = 160000
D = 256
H = 512
C = 40

N_ACC = 10240
E_PAD = 163840
CHUNK = 128
ROWS_PER_TILE = N_ACC // 16


def _sc_agg(p_slices, edge_split):
    n_out = 2 if edge_split else p_slices
    mesh = plsc.VectorSubcoreMesh(core_axis_name="c", subcore_axis_name="s")

    @functools.partial(
        pl.kernel, mesh=mesh,
        out_type=[jax.ShapeDtypeStruct((n_out, N_ACC, 128), jnp.float32)],
        scratch_types=[
            pltpu.VMEM((CHUNK,), jnp.int32),
            pltpu.VMEM((CHUNK,), jnp.int32),
            pltpu.VMEM((CHUNK, 128), jnp.float32),
            pltpu.VMEM_SHARED((N_ACC, 128), jnp.float32),
            pltpu.SemaphoreType.DMA,
        ])
    def agg_kernel(table_hbm, idxs_hbm, dst_hbm, zrows_hbm, out_hbm,
                   idx_v, dst_v, rows_v, acc_sh, sem):
        cid = lax.axis_index("c")
        tid = lax.axis_index("s")
        row0 = tid * ROWS_PER_TILE

        if edge_split:
            per_tile = E_PAD // 32
            tile_base = cid * (E_PAD // 2) + tid * per_tile
            passes = 1
        else:
            per_tile = E_PAD // 16
            tile_base = tid * per_tile
            passes = p_slices // 2
        n_chunks = per_tile // CHUNK

        for pp in range(passes):
            q = cid if edge_split else pp * 2 + cid
            qi = 0 if edge_split else q

            pltpu.sync_copy(zrows_hbm.at[pl.ds(row0, ROWS_PER_TILE)],
                            acc_sh.at[pl.ds(row0, ROWS_PER_TILE)])
            plsc.subcore_barrier()

            def chunk_body(g, carry):
                off = tile_base + g * CHUNK
                pltpu.sync_copy(idxs_hbm.at[qi, pl.ds(off, CHUNK)], idx_v)
                pltpu.sync_copy(dst_hbm.at[pl.ds(off, CHUNK)], dst_v)
                pltpu.async_copy(table_hbm.at[idx_v], rows_v, sem).wait()
                pltpu.sync_copy(rows_v, acc_sh.at[dst_v], add=True)
                return carry

            lax.fori_loop(0, n_chunks, chunk_body, 0)
            plsc.subcore_barrier()

            pltpu.sync_copy(acc_sh.at[pl.ds(row0, ROWS_PER_TILE)],
                            out_hbm.at[q, pl.ds(row0, ROWS_PER_TILE)])
            plsc.subcore_barrier()

    return agg_kernel


_agg_cols2 = _sc_agg(p_slices=2, edge_split=False)
_agg_cols4 = _sc_agg(p_slices=4, edge_split=False)
_agg_edges = _sc_agg(p_slices=1, edge_split=True)


BN = 400


def _tc1_body(x_ref, agg_ref, degp_ref, ws_ref, wn_ref, b_ref, h_ref, inv_ref):
    deg = degp_ref[0][:, :1] + degp_ref[1][:, :1]
    invc = 1.0 / jnp.maximum(deg, 1.0)
    inv_ref[...] = invc
    acc = jnp.dot(x_ref[...], ws_ref[...], preferred_element_type=jnp.float32)
    for qq in range(2):
        acc += jnp.dot(agg_ref[qq] * invc, wn_ref[qq * 128:(qq + 1) * 128, :],
                       preferred_element_type=jnp.float32)
    acc += b_ref[...]
    h_ref[...] = jnp.maximum(acc, 0.0)


def _tc2_body(h1_ref, agg_ref, inv_ref, ws_ref, wn_ref, wn3_ref, b_ref,
              h2_ref, z_ref):
    invc = inv_ref[...]
    acc = jnp.dot(h1_ref[...], ws_ref[...], preferred_element_type=jnp.float32)
    for qq in range(4):
        acc += jnp.dot(agg_ref[qq] * invc, wn_ref[qq * 128:(qq + 1) * 128, :],
                       preferred_element_type=jnp.float32)
    acc += b_ref[...]
    h2 = jnp.maximum(acc, 0.0)
    h2_ref[...] = h2
    z_ref[...] = jnp.dot(h2, wn3_ref[...], preferred_element_type=jnp.float32)


def _tc3_body(h2_ref, agg_ref, inv_ref, ws_ref, b_ref, out_ref):
    invc = inv_ref[...]
    s = jnp.dot(h2_ref[...], ws_ref[...], preferred_element_type=jnp.float32)
    s += b_ref[...]
    s += (agg_ref[0] + agg_ref[1]) * invc
    col = lax.broadcasted_iota(jnp.int32, (BN, 128), 1)
    mask = col < C
    neg = jnp.float32(-1e30)
    m = jnp.max(jnp.where(mask, s, neg), axis=1, keepdims=True)
    e = jnp.where(mask, jnp.exp(s - m), 0.0)
    lse = jnp.log(jnp.sum(e, axis=1, keepdims=True))
    out_ref[...] = s - m - lse


def kernel(x, edge_index, Ws1, Wn1, b1, Ws2, Wn2, b2, Ws3, Wn3, b3):
    src = edge_index[0].astype(jnp.int32)
    dst = edge_index[1].astype(jnp.int32)
    pad = E_PAD - E
    src_p = jnp.concatenate([src, jnp.zeros((pad,), jnp.int32)])
    dst_p = jnp.concatenate([dst, jnp.full((pad,), N, jnp.int32)])
    z128 = jnp.zeros((N_ACC, 128), jnp.float32)
    ones_tab = jnp.ones((8, 128), jnp.float32)
    zidx = jnp.zeros((1, E_PAD), jnp.int32)

    nb = N // BN

    (degp,) = _agg_edges(ones_tab, zidx, dst_p, z128)

    idxs1 = jnp.stack([src_p * 2, src_p * 2 + 1])
    (agg1,) = _agg_cols2(x.reshape(N * 2, 128), idxs1, dst_p, z128)
    h1, inv = pl.pallas_call(
        _tc1_body,
        grid=(nb,),
        in_specs=[
            pl.BlockSpec((BN, D), lambda i: (i, 0)),
            pl.BlockSpec((2, BN, 128), lambda i: (0, i, 0)),
            pl.BlockSpec((2, BN, 128), lambda i: (0, i, 0)),
            pl.BlockSpec((D, H), lambda i: (0, 0)),
            pl.BlockSpec((D, H), lambda i: (0, 0)),
            pl.BlockSpec((1, H), lambda i: (0, 0)),
        ],
        out_specs=[
            pl.BlockSpec((BN, H), lambda i: (i, 0)),
            pl.BlockSpec((BN, 1), lambda i: (i, 0)),
        ],
        out_shape=[
            jax.ShapeDtypeStruct((N, H), jnp.float32),
            jax.ShapeDtypeStruct((N, 1), jnp.float32),
        ],
    )(x, agg1, degp, Ws1, Wn1, b1.reshape(1, H))

    idxs2 = jnp.stack([src_p * 4 + qq for qq in range(4)])
    (agg2,) = _agg_cols4(h1.reshape(N * 4, 128), idxs2, dst_p, z128)
    wn3p = jnp.pad(Wn3, ((0, 0), (0, 128 - C)))
    h2, z = pl.pallas_call(
        _tc2_body,
        grid=(nb,),
        in_specs=[
            pl.BlockSpec((BN, H), lambda i: (i, 0)),
            pl.BlockSpec((4, BN, 128), lambda i: (0, i, 0)),
            pl.BlockSpec((BN, 1), lambda i: (i, 0)),
            pl.BlockSpec((H, H), lambda i: (0, 0)),
            pl.BlockSpec((H, H), lambda i: (0, 0)),
            pl.BlockSpec((H, 128), lambda i: (0, 0)),
            pl.BlockSpec((1, H), lambda i: (0, 0)),
        ],
        out_specs=[
            pl.BlockSpec((BN, H), lambda i: (i, 0)),
            pl.BlockSpec((BN, 128), lambda i: (i, 0)),
        ],
        out_shape=[
            jax.ShapeDtypeStruct((N, H), jnp.float32),
            jax.ShapeDtypeStruct((N, 128), jnp.float32),
        ],
    )(h1, agg2, inv, Ws2, Wn2, wn3p, b2.reshape(1, H))

    (agg3,) = _agg_edges(z, src_p[None], dst_p, z128)
    ws3p = jnp.pad(Ws3, ((0, 0), (0, 128 - C)))
    b3p = jnp.pad(b3, ((0, 128 - C))).reshape(1, 128)
    out = pl.pallas_call(
        _tc3_body,
        grid=(nb,),
        in_specs=[
            pl.BlockSpec((BN, H), lambda i: (i, 0)),
            pl.BlockSpec((2, BN, 128), lambda i: (0, i, 0)),
            pl.BlockSpec((BN, 1), lambda i: (i, 0)),
            pl.BlockSpec((H, 128), lambda i: (0, 0)),
            pl.BlockSpec((1, 128), lambda i: (0, 0)),
        ],
        out_specs=pl.BlockSpec((BN, 128), lambda i: (i, 0)),
        out_shape=jax.ShapeDtypeStruct((N, 128), jnp.float32),
    )(h2, agg3, inv, ws3p, b3p)
    return out[:, :C]


# speedup vs baseline: 4.4440x; 4.4440x over previous
import functools

import jax
import jax.numpy as jnp
from jax import lax
from jax.experimental import pallas as pl
from jax.experimental.pallas import tpu as pltpu
from jax.experimental.pallas import tpu_sc as plsc

N = 10000
E = 160000
D = 256
H = 512
C = 40

N_ACC = 10240
E_PAD = 163840
CHUNK = 128
ROWS_PER_TILE = N_ACC // 16


def _sc_agg(p_slices, edge_split):
    n_out = 2 if edge_split else p_slices
    mesh = plsc.VectorSubcoreMesh(core_axis_name="c", subcore_axis_name="s")

    @functools.partial(
        pl.kernel, mesh=mesh,
        out_type=[jax.ShapeDtypeStruct((n_out, N_ACC, 128), jnp.float32)],
        scratch_types=[
            pltpu.VMEM((CHUNK,), jnp.int32),
            pltpu.VMEM((CHUNK,), jnp.int32),
            pltpu.VMEM((CHUNK, 128), jnp.float32),
            pltpu.VMEM_SHARED((N_ACC, 128), jnp.float32),
            pltpu.SemaphoreType.DMA,
        ])
    def agg_kernel(table_hbm, idxs_hbm, dst_hbm, zrows_hbm, out_hbm,
                   idx_v, dst_v, rows_v, acc_sh, sem):
        cid = lax.axis_index("c")
        tid = lax.axis_index("s")
        row0 = tid * ROWS_PER_TILE

        if edge_split:
            per_tile = E_PAD // 32
            tile_base = cid * (E_PAD // 2) + tid * per_tile
            passes = 1
        else:
            per_tile = E_PAD // 16
            tile_base = tid * per_tile
            passes = p_slices // 2
        n_chunks = per_tile // CHUNK

        for pp in range(passes):
            q = cid if edge_split else pp * 2 + cid
            qi = 0 if edge_split else q

            pltpu.sync_copy(zrows_hbm.at[pl.ds(row0, ROWS_PER_TILE)],
                            acc_sh.at[pl.ds(row0, ROWS_PER_TILE)])
            plsc.subcore_barrier()

            def chunk_body(g, carry):
                off = tile_base + g * CHUNK
                pltpu.sync_copy(idxs_hbm.at[qi, pl.ds(off, CHUNK)], idx_v)
                pltpu.sync_copy(dst_hbm.at[pl.ds(off, CHUNK)], dst_v)
                pltpu.async_copy(table_hbm.at[idx_v], rows_v, sem).wait()
                pltpu.sync_copy(rows_v, acc_sh.at[dst_v], add=True)
                return carry

            lax.fori_loop(0, n_chunks, chunk_body, 0)
            plsc.subcore_barrier()

            pltpu.sync_copy(acc_sh.at[pl.ds(row0, ROWS_PER_TILE)],
                            out_hbm.at[q, pl.ds(row0, ROWS_PER_TILE)])
            plsc.subcore_barrier()

    return agg_kernel


_agg_cols2 = _sc_agg(p_slices=2, edge_split=False)
_agg_cols4 = _sc_agg(p_slices=4, edge_split=False)
_agg_edges = _sc_agg(p_slices=1, edge_split=True)


@functools.partial(
    pl.kernel,
    mesh=plsc.VectorSubcoreMesh(core_axis_name="c", subcore_axis_name="s"),
    out_type=[jax.ShapeDtypeStruct((2, N_ACC, 128), jnp.float32)],
    scratch_types=[
        pltpu.VMEM((CHUNK,), jnp.int32),
        pltpu.VMEM((CHUNK, 128), jnp.float32),
        pltpu.VMEM_SHARED((N_ACC, 128), jnp.float32),
    ])
def _deg_kernel(ones_hbm, dst_hbm, zrows_hbm, out_hbm,
                dst_v, ones_v, acc_sh):
    cid = lax.axis_index("c")
    tid = lax.axis_index("s")
    row0 = tid * ROWS_PER_TILE
    per_tile = E_PAD // 32
    tile_base = cid * (E_PAD // 2) + tid * per_tile

    pltpu.sync_copy(ones_hbm, ones_v)
    pltpu.sync_copy(zrows_hbm.at[pl.ds(row0, ROWS_PER_TILE)],
                    acc_sh.at[pl.ds(row0, ROWS_PER_TILE)])
    plsc.subcore_barrier()

    def chunk_body(g, carry):
        pltpu.sync_copy(dst_hbm.at[pl.ds(tile_base + g * CHUNK, CHUNK)], dst_v)
        pltpu.sync_copy(ones_v, acc_sh.at[dst_v], add=True)
        return carry

    lax.fori_loop(0, per_tile // CHUNK, chunk_body, 0)
    plsc.subcore_barrier()
    pltpu.sync_copy(acc_sh.at[pl.ds(row0, ROWS_PER_TILE)],
                    out_hbm.at[cid, pl.ds(row0, ROWS_PER_TILE)])


BN = 400


def _tc1_body(x_ref, agg_ref, degp_ref, ws_ref, wn_ref, b_ref, h_ref, inv_ref):
    deg = degp_ref[0][:, :1] + degp_ref[1][:, :1]
    invc = 1.0 / jnp.maximum(deg, 1.0)
    inv_ref[...] = invc
    acc = jnp.dot(x_ref[...], ws_ref[...], preferred_element_type=jnp.float32)
    for qq in range(2):
        acc += jnp.dot(agg_ref[qq] * invc, wn_ref[qq * 128:(qq + 1) * 128, :],
                       preferred_element_type=jnp.float32)
    acc += b_ref[...]
    h_ref[...] = jnp.maximum(acc, 0.0)


def _tc2_body(h1_ref, agg_ref, inv_ref, ws_ref, wn_ref, wn3_ref, b_ref,
              h2_ref, z_ref):
    invc = inv_ref[...]
    acc = jnp.dot(h1_ref[...], ws_ref[...], preferred_element_type=jnp.float32)
    for qq in range(4):
        acc += jnp.dot(agg_ref[qq] * invc, wn_ref[qq * 128:(qq + 1) * 128, :],
                       preferred_element_type=jnp.float32)
    acc += b_ref[...]
    h2 = jnp.maximum(acc, 0.0)
    h2_ref[...] = h2
    z_ref[...] = jnp.dot(h2, wn3_ref[...], preferred_element_type=jnp.float32)


def _tc3_body(h2_ref, agg_ref, inv_ref, ws_ref, b_ref, out_ref):
    invc = inv_ref[...]
    s = jnp.dot(h2_ref[...], ws_ref[...], preferred_element_type=jnp.float32)
    s += b_ref[...]
    s += (agg_ref[0] + agg_ref[1]) * invc
    col = lax.broadcasted_iota(jnp.int32, (BN, 128), 1)
    mask = col < C
    neg = jnp.float32(-1e30)
    m = jnp.max(jnp.where(mask, s, neg), axis=1, keepdims=True)
    e = jnp.where(mask, jnp.exp(s - m), 0.0)
    lse = jnp.log(jnp.sum(e, axis=1, keepdims=True))
    out_ref[...] = s - m - lse


def kernel(x, edge_index, Ws1, Wn1, b1, Ws2, Wn2, b2, Ws3, Wn3, b3):
    src = edge_index[0].astype(jnp.int32)
    dst = edge_index[1].astype(jnp.int32)
    pad = E_PAD - E
    src_p = jnp.concatenate([src, jnp.zeros((pad,), jnp.int32)])
    dst_p = jnp.concatenate([dst, jnp.full((pad,), N, jnp.int32)])
    z128 = jnp.zeros((N_ACC, 128), jnp.float32)
    ones_tab = jnp.ones((CHUNK, 128), jnp.float32)

    nb = N // BN

    (degp,) = _deg_kernel(ones_tab, dst_p, z128)

    idxs1 = jnp.stack([src_p * 2, src_p * 2 + 1])
    (agg1,) = _agg_cols2(x.reshape(N * 2, 128), idxs1, dst_p, z128)
    h1, inv = pl.pallas_call(
        _tc1_body,
        grid=(nb,),
        in_specs=[
            pl.BlockSpec((BN, D), lambda i: (i, 0)),
            pl.BlockSpec((2, BN, 128), lambda i: (0, i, 0)),
            pl.BlockSpec((2, BN, 128), lambda i: (0, i, 0)),
            pl.BlockSpec((D, H), lambda i: (0, 0)),
            pl.BlockSpec((D, H), lambda i: (0, 0)),
            pl.BlockSpec((1, H), lambda i: (0, 0)),
        ],
        out_specs=[
            pl.BlockSpec((BN, H), lambda i: (i, 0)),
            pl.BlockSpec((BN, 1), lambda i: (i, 0)),
        ],
        out_shape=[
            jax.ShapeDtypeStruct((N, H), jnp.float32),
            jax.ShapeDtypeStruct((N, 1), jnp.float32),
        ],
    )(x, agg1, degp, Ws1, Wn1, b1.reshape(1, H))

    idxs2 = jnp.stack([src_p * 4 + qq for qq in range(4)])
    (agg2,) = _agg_cols4(h1.reshape(N * 4, 128), idxs2, dst_p, z128)
    wn3p = jnp.pad(Wn3, ((0, 0), (0, 128 - C)))
    h2, z = pl.pallas_call(
        _tc2_body,
        grid=(nb,),
        in_specs=[
            pl.BlockSpec((BN, H), lambda i: (i, 0)),
            pl.BlockSpec((4, BN, 128), lambda i: (0, i, 0)),
            pl.BlockSpec((BN, 1), lambda i: (i, 0)),
            pl.BlockSpec((H, H), lambda i: (0, 0)),
            pl.BlockSpec((H, H), lambda i: (0, 0)),
            pl.BlockSpec((H, 128), lambda i: (0, 0)),
            pl.BlockSpec((1, H), lambda i: (0, 0)),
        ],
        out_specs=[
            pl.BlockSpec((BN, H), lambda i: (i, 0)),
            pl.BlockSpec((BN, 128), lambda i: (i, 0)),
        ],
        out_shape=[
            jax.ShapeDtypeStruct((N, H), jnp.float32),
            jax.ShapeDtypeStruct((N, 128), jnp.float32),
        ],
    )(h1, agg2, inv, Ws2, Wn2, wn3p, b2.reshape(1, H))

    (agg3,) = _agg_edges(z, src_p[None], dst_p, z128)
    ws3p = jnp.pad(Ws3, ((0, 0), (0, 128 - C)))
    b3p = jnp.pad(b3, ((0, 128 - C))).reshape(1, 128)
    out = pl.pallas_call(
        _tc3_body,
        grid=(nb,),
        in_specs=[
            pl.BlockSpec((BN, H), lambda i: (i, 0)),
            pl.BlockSpec((2, BN, 128), lambda i: (0, i, 0)),
            pl.BlockSpec((BN, 1), lambda i: (i, 0)),
            pl.BlockSpec((H, 128), lambda i: (0, 0)),
            pl.BlockSpec((1, 128), lambda i: (0, 0)),
        ],
        out_specs=pl.BlockSpec((BN, 128), lambda i: (i, 0)),
        out_shape=jax.ShapeDtypeStruct((N, 128), jnp.float32),
    )(h2, agg3, inv, ws3p, b3p)
    return out[:, :C]


# speedup vs baseline: 5.6222x; 1.2651x over previous
import functools

import jax
import jax.numpy as jnp
from jax import lax
from jax.experimental import pallas as pl
from jax.experimental.pallas import tpu as pltpu
from jax.experimental.pallas import tpu_sc as plsc

N = 10000
E = 160000
D = 256
H = 512
C = 40

N_ACC = 10240
E_PAD = 163840
CHUNK = 128
E_ALLOC = E_PAD + CHUNK
ROWS_PER_TILE = N_ACC // 16


def _sc_agg(p_slices, edge_split):
    n_out = 2 if edge_split else p_slices
    mesh = plsc.VectorSubcoreMesh(core_axis_name="c", subcore_axis_name="s")

    @functools.partial(
        pl.kernel, mesh=mesh,
        out_type=[jax.ShapeDtypeStruct((n_out, N_ACC, 128), jnp.float32)],
        scratch_types=[
            pltpu.VMEM((2, CHUNK), jnp.int32),
            pltpu.VMEM((2, CHUNK), jnp.int32),
            pltpu.VMEM((2, CHUNK, 128), jnp.float32),
            pltpu.VMEM_SHARED((N_ACC, 128), jnp.float32),
            pltpu.SemaphoreType.DMA,
            pltpu.SemaphoreType.DMA,
        ])
    def agg_kernel(table_hbm, idxs_hbm, dst_hbm, zrows_hbm, out_hbm,
                   idx_v, dst_v, rows_v, acc_sh, sem0, sem1):
        cid = lax.axis_index("c")
        tid = lax.axis_index("s")
        row0 = tid * ROWS_PER_TILE

        if edge_split:
            per_tile = E_PAD // 32
            tile_base = cid * (E_PAD // 2) + tid * per_tile
            passes = 1
        else:
            per_tile = E_PAD // 16
            tile_base = tid * per_tile
            passes = p_slices // 2
        n_chunks = per_tile // CHUNK
        sems = (sem0, sem1)

        for pp in range(passes):
            q = cid if edge_split else pp * 2 + cid
            qi = 0 if edge_split else q

            def load(c, b):
                off = tile_base + c * CHUNK
                pltpu.sync_copy(idxs_hbm.at[qi, pl.ds(off, CHUNK)],
                                idx_v.at[b])
                pltpu.sync_copy(dst_hbm.at[pl.ds(off, CHUNK)], dst_v.at[b])

            def start_gather(b):
                pltpu.async_copy(table_hbm.at[idx_v.at[b]], rows_v.at[b],
                                 sems[b])

            def wait_gather(b):
                pltpu.make_async_copy(table_hbm.at[idx_v.at[b]],
                                      rows_v.at[b], sems[b]).wait()

            pltpu.sync_copy(zrows_hbm.at[pl.ds(row0, ROWS_PER_TILE)],
                            acc_sh.at[pl.ds(row0, ROWS_PER_TILE)])
            load(0, 0)
            start_gather(0)
            plsc.subcore_barrier()

            def chunk_body(g2, carry):
                load(2 * g2 + 1, 1)
                start_gather(1)
                wait_gather(0)
                pltpu.sync_copy(rows_v.at[0], acc_sh.at[dst_v.at[0]],
                                add=True)
                load(2 * g2 + 2, 0)
                start_gather(0)
                wait_gather(1)
                pltpu.sync_copy(rows_v.at[1], acc_sh.at[dst_v.at[1]],
                                add=True)
                return carry

            lax.fori_loop(0, n_chunks // 2, chunk_body, 0)
            wait_gather(0)
            plsc.subcore_barrier()

            pltpu.sync_copy(acc_sh.at[pl.ds(row0, ROWS_PER_TILE)],
                            out_hbm.at[q, pl.ds(row0, ROWS_PER_TILE)])
            plsc.subcore_barrier()

    return agg_kernel


_agg_cols2 = _sc_agg(p_slices=2, edge_split=False)
_agg_cols4 = _sc_agg(p_slices=4, edge_split=False)
_agg_edges = _sc_agg(p_slices=1, edge_split=True)


@functools.partial(
    pl.kernel,
    mesh=plsc.VectorSubcoreMesh(core_axis_name="c", subcore_axis_name="s"),
    out_type=[jax.ShapeDtypeStruct((2, N_ACC, 128), jnp.float32)],
    scratch_types=[
        pltpu.VMEM((CHUNK,), jnp.int32),
        pltpu.VMEM((CHUNK, 128), jnp.float32),
        pltpu.VMEM_SHARED((N_ACC, 128), jnp.float32),
    ])
def _deg_kernel(ones_hbm, dst_hbm, zrows_hbm, out_hbm,
                dst_v, ones_v, acc_sh):
    cid = lax.axis_index("c")
    tid = lax.axis_index("s")
    row0 = tid * ROWS_PER_TILE
    per_tile = E_PAD // 32
    tile_base = cid * (E_PAD // 2) + tid * per_tile

    pltpu.sync_copy(ones_hbm, ones_v)
    pltpu.sync_copy(zrows_hbm.at[pl.ds(row0, ROWS_PER_TILE)],
                    acc_sh.at[pl.ds(row0, ROWS_PER_TILE)])
    plsc.subcore_barrier()

    def chunk_body(g, carry):
        pltpu.sync_copy(dst_hbm.at[pl.ds(tile_base + g * CHUNK, CHUNK)], dst_v)
        pltpu.sync_copy(ones_v, acc_sh.at[dst_v], add=True)
        return carry

    lax.fori_loop(0, per_tile // CHUNK, chunk_body, 0)
    plsc.subcore_barrier()
    pltpu.sync_copy(acc_sh.at[pl.ds(row0, ROWS_PER_TILE)],
                    out_hbm.at[cid, pl.ds(row0, ROWS_PER_TILE)])


BN = 400


def _tc1_body(x_ref, agg_ref, degp_ref, ws_ref, wn_ref, b_ref, h_ref, inv_ref):
    deg = degp_ref[0][:, :1] + degp_ref[1][:, :1]
    invc = 1.0 / jnp.maximum(deg, 1.0)
    inv_ref[...] = invc
    acc = jnp.dot(x_ref[...], ws_ref[...], preferred_element_type=jnp.float32)
    for qq in range(2):
        acc += jnp.dot(agg_ref[qq] * invc, wn_ref[qq * 128:(qq + 1) * 128, :],
                       preferred_element_type=jnp.float32)
    acc += b_ref[...]
    h_ref[...] = jnp.maximum(acc, 0.0)


def _tc2_body(h1_ref, agg_ref, inv_ref, ws_ref, wn_ref, wn3_ref, b_ref,
              h2_ref, z_ref):
    invc = inv_ref[...]
    acc = jnp.dot(h1_ref[...], ws_ref[...], preferred_element_type=jnp.float32)
    for qq in range(4):
        acc += jnp.dot(agg_ref[qq] * invc, wn_ref[qq * 128:(qq + 1) * 128, :],
                       preferred_element_type=jnp.float32)
    acc += b_ref[...]
    h2 = jnp.maximum(acc, 0.0)
    h2_ref[...] = h2
    z_ref[...] = jnp.dot(h2, wn3_ref[...], preferred_element_type=jnp.float32)


def _tc3_body(h2_ref, agg_ref, inv_ref, ws_ref, b_ref, out_ref):
    invc = inv_ref[...]
    s = jnp.dot(h2_ref[...], ws_ref[...], preferred_element_type=jnp.float32)
    s += b_ref[...]
    s += (agg_ref[0] + agg_ref[1]) * invc
    col = lax.broadcasted_iota(jnp.int32, (BN, 128), 1)
    mask = col < C
    neg = jnp.float32(-1e30)
    m = jnp.max(jnp.where(mask, s, neg), axis=1, keepdims=True)
    e = jnp.where(mask, jnp.exp(s - m), 0.0)
    lse = jnp.log(jnp.sum(e, axis=1, keepdims=True))
    out_ref[...] = s - m - lse


def kernel(x, edge_index, Ws1, Wn1, b1, Ws2, Wn2, b2, Ws3, Wn3, b3):
    src = edge_index[0].astype(jnp.int32)
    dst = edge_index[1].astype(jnp.int32)
    pad = E_ALLOC - E
    src_p = jnp.concatenate([src, jnp.zeros((pad,), jnp.int32)])
    dst_p = jnp.concatenate([dst, jnp.full((pad,), N, jnp.int32)])
    z128 = jnp.zeros((N_ACC, 128), jnp.float32)
    ones_tab = jnp.ones((CHUNK, 128), jnp.float32)

    nb = N // BN

    (degp,) = _deg_kernel(ones_tab, dst_p, z128)

    idxs1 = jnp.stack([src_p * 2, src_p * 2 + 1])
    (agg1,) = _agg_cols2(x.reshape(N * 2, 128), idxs1, dst_p, z128)
    h1, inv = pl.pallas_call(
        _tc1_body,
        grid=(nb,),
        in_specs=[
            pl.BlockSpec((BN, D), lambda i: (i, 0)),
            pl.BlockSpec((2, BN, 128), lambda i: (0, i, 0)),
            pl.BlockSpec((2, BN, 128), lambda i: (0, i, 0)),
            pl.BlockSpec((D, H), lambda i: (0, 0)),
            pl.BlockSpec((D, H), lambda i: (0, 0)),
            pl.BlockSpec((1, H), lambda i: (0, 0)),
        ],
        out_specs=[
            pl.BlockSpec((BN, H), lambda i: (i, 0)),
            pl.BlockSpec((BN, 1), lambda i: (i, 0)),
        ],
        out_shape=[
            jax.ShapeDtypeStruct((N, H), jnp.float32),
            jax.ShapeDtypeStruct((N, 1), jnp.float32),
        ],
    )(x, agg1, degp, Ws1, Wn1, b1.reshape(1, H))

    idxs2 = jnp.stack([src_p * 4 + qq for qq in range(4)])
    (agg2,) = _agg_cols4(h1.reshape(N * 4, 128), idxs2, dst_p, z128)
    wn3p = jnp.pad(Wn3, ((0, 0), (0, 128 - C)))
    h2, z = pl.pallas_call(
        _tc2_body,
        grid=(nb,),
        in_specs=[
            pl.BlockSpec((BN, H), lambda i: (i, 0)),
            pl.BlockSpec((4, BN, 128), lambda i: (0, i, 0)),
            pl.BlockSpec((BN, 1), lambda i: (i, 0)),
            pl.BlockSpec((H, H), lambda i: (0, 0)),
            pl.BlockSpec((H, H), lambda i: (0, 0)),
            pl.BlockSpec((H, 128), lambda i: (0, 0)),
            pl.BlockSpec((1, H), lambda i: (0, 0)),
        ],
        out_specs=[
            pl.BlockSpec((BN, H), lambda i: (i, 0)),
            pl.BlockSpec((BN, 128), lambda i: (i, 0)),
        ],
        out_shape=[
            jax.ShapeDtypeStruct((N, H), jnp.float32),
            jax.ShapeDtypeStruct((N, 128), jnp.float32),
        ],
    )(h1, agg2, inv, Ws2, Wn2, wn3p, b2.reshape(1, H))

    (agg3,) = _agg_edges(z, src_p[None], dst_p, z128)
    ws3p = jnp.pad(Ws3, ((0, 0), (0, 128 - C)))
    b3p = jnp.pad(b3, ((0, 128 - C))).reshape(1, 128)
    out = pl.pallas_call(
        _tc3_body,
        grid=(nb,),
        in_specs=[
            pl.BlockSpec((BN, H), lambda i: (i, 0)),
            pl.BlockSpec((2, BN, 128), lambda i: (0, i, 0)),
            pl.BlockSpec((BN, 1), lambda i: (i, 0)),
            pl.BlockSpec((H, 128), lambda i: (0, 0)),
            pl.BlockSpec((1, 128), lambda i: (0, 0)),
        ],
        out_specs=pl.BlockSpec((BN, 128), lambda i: (i, 0)),
        out_shape=jax.ShapeDtypeStruct((N, 128), jnp.float32),
    )(h2, agg3, inv, ws3p, b3p)
    return out[:, :C]


# speedup vs baseline: 5.7974x; 1.0312x over previous
import functools

import jax
import jax.numpy as jnp
from jax import lax
from jax.experimental import pallas as pl
from jax.experimental.pallas import tpu as pltpu
from jax.experimental.pallas import tpu_sc as plsc

N = 10000
E = 160000
D = 256
H = 512
C = 40

N_ACC = 10240
E_PAD = 163840
CHUNK = 128
NBUF = 2
E_ALLOC = E_PAD + NBUF * CHUNK
ROWS_PER_TILE = N_ACC // 16


def _sc_agg(p_slices, edge_split):
    n_out = 2 if edge_split else p_slices
    mesh = plsc.VectorSubcoreMesh(core_axis_name="c", subcore_axis_name="s")

    @functools.partial(
        pl.kernel, mesh=mesh,
        out_type=[jax.ShapeDtypeStruct((n_out, N_ACC, 128), jnp.float32)],
        scratch_types=[
            pltpu.VMEM((NBUF, CHUNK), jnp.int32),
            pltpu.VMEM((NBUF, CHUNK), jnp.int32),
            pltpu.VMEM((NBUF, CHUNK, 128), jnp.float32),
            pltpu.VMEM_SHARED((N_ACC, 128), jnp.float32),
        ] + [pltpu.SemaphoreType.DMA] * (2 * NBUF))
    def agg_kernel(table_hbm, idxs_hbm, dst_hbm, zrows_hbm, out_hbm,
                   idx_v, dst_v, rows_v, acc_sh, *sems):
        gsem = sems[:NBUF]
        lsem = sems[NBUF:]
        cid = lax.axis_index("c")
        tid = lax.axis_index("s")
        row0 = tid * ROWS_PER_TILE

        if edge_split:
            per_tile = E_PAD // 32
            tile_base = cid * (E_PAD // 2) + tid * per_tile
            passes = 1
        else:
            per_tile = E_PAD // 16
            tile_base = tid * per_tile
            passes = p_slices // 2
        n_chunks = per_tile // CHUNK
        assert n_chunks % NBUF == 0

        for pp in range(passes):
            q = cid if edge_split else pp * 2 + cid
            qi = 0 if edge_split else q

            def load_async(c, b):
                off = tile_base + c * CHUNK
                pltpu.async_copy(idxs_hbm.at[qi, pl.ds(off, CHUNK)],
                                 idx_v.at[b], lsem[b])
                pltpu.async_copy(dst_hbm.at[pl.ds(off, CHUNK)],
                                 dst_v.at[b], lsem[b])

            def wait_load(c, b):
                off = tile_base + c * CHUNK
                pltpu.make_async_copy(idxs_hbm.at[qi, pl.ds(off, CHUNK)],
                                      idx_v.at[b], lsem[b]).wait()
                pltpu.make_async_copy(dst_hbm.at[pl.ds(off, CHUNK)],
                                      dst_v.at[b], lsem[b]).wait()

            def start_gather(b):
                pltpu.async_copy(table_hbm.at[idx_v.at[b]], rows_v.at[b],
                                 gsem[b])

            def wait_gather(b):
                pltpu.make_async_copy(table_hbm.at[idx_v.at[b]],
                                      rows_v.at[b], gsem[b]).wait()

            pltpu.sync_copy(zrows_hbm.at[pl.ds(row0, ROWS_PER_TILE)],
                            acc_sh.at[pl.ds(row0, ROWS_PER_TILE)])
            for b in range(NBUF):
                load_async(b, b)
            wait_load(0, 0)
            start_gather(0)
            plsc.subcore_barrier()

            def chunk_body(g4, carry):
                c0 = g4 * NBUF
                for j in range(NBUF):
                    c = c0 + j
                    nb = (j + 1) % NBUF
                    wait_load(c + 1, nb)
                    start_gather(nb)
                    wait_gather(j)
                    pltpu.sync_copy(rows_v.at[j], acc_sh.at[dst_v.at[j]],
                                    add=True)
                    load_async(c + NBUF, j)
                return carry

            lax.fori_loop(0, n_chunks // NBUF, chunk_body, 0)
            wait_gather(0)
            for b in range(1, NBUF):
                wait_load(n_chunks + b, b)
            plsc.subcore_barrier()

            pltpu.sync_copy(acc_sh.at[pl.ds(row0, ROWS_PER_TILE)],
                            out_hbm.at[q, pl.ds(row0, ROWS_PER_TILE)])
            plsc.subcore_barrier()

    return agg_kernel


_agg_cols2 = _sc_agg(p_slices=2, edge_split=False)
_agg_cols4 = _sc_agg(p_slices=4, edge_split=False)
_agg_edges = _sc_agg(p_slices=1, edge_split=True)


@functools.partial(
    pl.kernel,
    mesh=plsc.VectorSubcoreMesh(core_axis_name="c", subcore_axis_name="s"),
    out_type=[jax.ShapeDtypeStruct((2, N_ACC, 128), jnp.float32)],
    scratch_types=[
        pltpu.VMEM((CHUNK,), jnp.int32),
        pltpu.VMEM((CHUNK, 128), jnp.float32),
        pltpu.VMEM_SHARED((N_ACC, 128), jnp.float32),
    ])
def _deg_kernel(ones_hbm, dst_hbm, zrows_hbm, out_hbm,
                dst_v, ones_v, acc_sh):
    cid = lax.axis_index("c")
    tid = lax.axis_index("s")
    row0 = tid * ROWS_PER_TILE
    per_tile = E_PAD // 32
    tile_base = cid * (E_PAD // 2) + tid * per_tile

    pltpu.sync_copy(ones_hbm, ones_v)
    pltpu.sync_copy(zrows_hbm.at[pl.ds(row0, ROWS_PER_TILE)],
                    acc_sh.at[pl.ds(row0, ROWS_PER_TILE)])
    plsc.subcore_barrier()

    def chunk_body(g, carry):
        pltpu.sync_copy(dst_hbm.at[pl.ds(tile_base + g * CHUNK, CHUNK)], dst_v)
        pltpu.sync_copy(ones_v, acc_sh.at[dst_v], add=True)
        return carry

    lax.fori_loop(0, per_tile // CHUNK, chunk_body, 0)
    plsc.subcore_barrier()
    pltpu.sync_copy(acc_sh.at[pl.ds(row0, ROWS_PER_TILE)],
                    out_hbm.at[cid, pl.ds(row0, ROWS_PER_TILE)])


BN = 400


def _tc1_body(x_ref, agg_ref, degp_ref, ws_ref, wn_ref, b_ref, h_ref, inv_ref):
    deg = degp_ref[0][:, :1] + degp_ref[1][:, :1]
    invc = 1.0 / jnp.maximum(deg, 1.0)
    inv_ref[...] = invc
    acc = jnp.dot(x_ref[...], ws_ref[...], preferred_element_type=jnp.float32)
    for qq in range(2):
        acc += jnp.dot(agg_ref[qq] * invc, wn_ref[qq * 128:(qq + 1) * 128, :],
                       preferred_element_type=jnp.float32)
    acc += b_ref[...]
    h_ref[...] = jnp.maximum(acc, 0.0)


def _tc2_body(h1_ref, agg_ref, inv_ref, ws_ref, wn_ref, wn3_ref, b_ref,
              h2_ref, z_ref):
    invc = inv_ref[...]
    acc = jnp.dot(h1_ref[...], ws_ref[...], preferred_element_type=jnp.float32)
    for qq in range(4):
        acc += jnp.dot(agg_ref[qq] * invc, wn_ref[qq * 128:(qq + 1) * 128, :],
                       preferred_element_type=jnp.float32)
    acc += b_ref[...]
    h2 = jnp.maximum(acc, 0.0)
    h2_ref[...] = h2
    z_ref[...] = jnp.dot(h2, wn3_ref[...], preferred_element_type=jnp.float32)


def _tc3_body(h2_ref, agg_ref, inv_ref, ws_ref, b_ref, out_ref):
    invc = inv_ref[...]
    s = jnp.dot(h2_ref[...], ws_ref[...], preferred_element_type=jnp.float32)
    s += b_ref[...]
    s += (agg_ref[0] + agg_ref[1]) * invc
    col = lax.broadcasted_iota(jnp.int32, (BN, 128), 1)
    mask = col < C
    neg = jnp.float32(-1e30)
    m = jnp.max(jnp.where(mask, s, neg), axis=1, keepdims=True)
    e = jnp.where(mask, jnp.exp(s - m), 0.0)
    lse = jnp.log(jnp.sum(e, axis=1, keepdims=True))
    out_ref[...] = s - m - lse


def kernel(x, edge_index, Ws1, Wn1, b1, Ws2, Wn2, b2, Ws3, Wn3, b3):
    src = edge_index[0].astype(jnp.int32)
    dst = edge_index[1].astype(jnp.int32)
    pad = E_ALLOC - E
    src_p = jnp.concatenate([src, jnp.zeros((pad,), jnp.int32)])
    dst_p = jnp.concatenate([dst, jnp.full((pad,), N, jnp.int32)])
    z128 = jnp.zeros((N_ACC, 128), jnp.float32)
    ones_tab = jnp.ones((CHUNK, 128), jnp.float32)

    nb = N // BN

    (degp,) = _deg_kernel(ones_tab, dst_p, z128)

    idxs1 = jnp.stack([src_p * 2, src_p * 2 + 1])
    (agg1,) = _agg_cols2(x.reshape(N * 2, 128), idxs1, dst_p, z128)
    h1, inv = pl.pallas_call(
        _tc1_body,
        grid=(nb,),
        in_specs=[
            pl.BlockSpec((BN, D), lambda i: (i, 0)),
            pl.BlockSpec((2, BN, 128), lambda i: (0, i, 0)),
            pl.BlockSpec((2, BN, 128), lambda i: (0, i, 0)),
            pl.BlockSpec((D, H), lambda i: (0, 0)),
            pl.BlockSpec((D, H), lambda i: (0, 0)),
            pl.BlockSpec((1, H), lambda i: (0, 0)),
        ],
        out_specs=[
            pl.BlockSpec((BN, H), lambda i: (i, 0)),
            pl.BlockSpec((BN, 1), lambda i: (i, 0)),
        ],
        out_shape=[
            jax.ShapeDtypeStruct((N, H), jnp.float32),
            jax.ShapeDtypeStruct((N, 1), jnp.float32),
        ],
    )(x, agg1, degp, Ws1, Wn1, b1.reshape(1, H))

    idxs2 = jnp.stack([src_p * 4 + qq for qq in range(4)])
    (agg2,) = _agg_cols4(h1.reshape(N * 4, 128), idxs2, dst_p, z128)
    wn3p = jnp.pad(Wn3, ((0, 0), (0, 128 - C)))
    h2, z = pl.pallas_call(
        _tc2_body,
        grid=(nb,),
        in_specs=[
            pl.BlockSpec((BN, H), lambda i: (i, 0)),
            pl.BlockSpec((4, BN, 128), lambda i: (0, i, 0)),
            pl.BlockSpec((BN, 1), lambda i: (i, 0)),
            pl.BlockSpec((H, H), lambda i: (0, 0)),
            pl.BlockSpec((H, H), lambda i: (0, 0)),
            pl.BlockSpec((H, 128), lambda i: (0, 0)),
            pl.BlockSpec((1, H), lambda i: (0, 0)),
        ],
        out_specs=[
            pl.BlockSpec((BN, H), lambda i: (i, 0)),
            pl.BlockSpec((BN, 128), lambda i: (i, 0)),
        ],
        out_shape=[
            jax.ShapeDtypeStruct((N, H), jnp.float32),
            jax.ShapeDtypeStruct((N, 128), jnp.float32),
        ],
    )(h1, agg2, inv, Ws2, Wn2, wn3p, b2.reshape(1, H))

    (agg3,) = _agg_edges(z, src_p[None], dst_p, z128)
    ws3p = jnp.pad(Ws3, ((0, 0), (0, 128 - C)))
    b3p = jnp.pad(b3, ((0, 128 - C))).reshape(1, 128)
    out = pl.pallas_call(
        _tc3_body,
        grid=(nb,),
        in_specs=[
            pl.BlockSpec((BN, H), lambda i: (i, 0)),
            pl.BlockSpec((2, BN, 128), lambda i: (0, i, 0)),
            pl.BlockSpec((BN, 1), lambda i: (i, 0)),
            pl.BlockSpec((H, 128), lambda i: (0, 0)),
            pl.BlockSpec((1, 128), lambda i: (0, 0)),
        ],
        out_specs=pl.BlockSpec((BN, 128), lambda i: (i, 0)),
        out_shape=jax.ShapeDtypeStruct((N, 128), jnp.float32),
    )(h2, agg3, inv, ws3p, b3p)
    return out[:, :C]
